# baseline (speedup 1.0000x reference)
import jax
import jax.numpy as jnp
from jax.experimental import pallas as pl
from jax.experimental.pallas import tpu as pltpu

_SAMPLES_PER_BLOCK = 1
_H_SPLIT = 4


def _gather_kernel(cam_ref, id_ref, w_ref, b_ref, ws_ref, bs_ref):
    bidx = pl.program_id(0)
    rem = id_ref[bidx] % 8
    sel = jax.lax.broadcasted_iota(jnp.int32, (8, 3), 0) == rem
    ws_ref[0] = jnp.sum(jnp.where(sel, w_ref[0], 0.0), axis=0, keepdims=True)
    bs_ref[0] = jnp.sum(jnp.where(sel, b_ref[0], 0.0), axis=0, keepdims=True)


def _make_affine(samples_per_block, n_chan):
    def _affine_kernel(ws_ref, bs_ref, img_ref, out_ref):
        bidx = pl.program_id(0)
        for s in range(samples_per_block):
            sample = bidx * samples_per_block + s
            for c in range(n_chan):
                wv = ws_ref[sample * n_chan + c]
                bv = bs_ref[sample * n_chan + c]
                out_ref[s, c] = img_ref[s, c] * wv + bv

    return _affine_kernel


def kernel(image, camindex, idindex, w, b):
    B, C, H, W = image.shape
    ws, bs = pl.pallas_call(
        _gather_kernel,
        grid_spec=pltpu.PrefetchScalarGridSpec(
            num_scalar_prefetch=2,
            grid=(B,),
            in_specs=[
                pl.BlockSpec((1, 8, 3), lambda bi, cam, idx: (cam[bi], idx[bi] // 8, 0)),
                pl.BlockSpec((1, 8, 3), lambda bi, cam, idx: (cam[bi], idx[bi] // 8, 0)),
            ],
            out_specs=[
                pl.BlockSpec((1, 1, 3), lambda bi, cam, idx: (bi, 0, 0)),
                pl.BlockSpec((1, 1, 3), lambda bi, cam, idx: (bi, 0, 0)),
            ],
        ),
        out_shape=[jax.ShapeDtypeStruct((B, 1, 3), jnp.float32)] * 2,
    )(camindex, idindex, w, b)

    sb = _SAMPLES_PER_BLOCK
    hb = H // _H_SPLIT
    ws_flat = ws.reshape(B * C)
    bs_flat = bs.reshape(B * C)
    out = pl.pallas_call(
        _make_affine(sb, C),
        grid_spec=pltpu.PrefetchScalarGridSpec(
            num_scalar_prefetch=2,
            grid=(B // sb, _H_SPLIT),
            in_specs=[
                pl.BlockSpec((sb, C, hb, W), lambda bi, hi, wsr, bsr: (bi, 0, hi, 0)),
            ],
            out_specs=pl.BlockSpec((sb, C, hb, W), lambda bi, hi, wsr, bsr: (bi, 0, hi, 0)),
        ),
        out_shape=jax.ShapeDtypeStruct(image.shape, image.dtype),
    )(ws_flat, bs_flat, image)
    return out


# speedup vs baseline: 1.0890x; 1.0890x over previous
import jax
import jax.numpy as jnp
from jax.experimental import pallas as pl
from jax.experimental.pallas import tpu as pltpu

_SAMPLES_PER_BLOCK = 4
_H_SPLIT = 1


def _gather_kernel(cam_ref, id_ref, w_ref, b_ref, ws_ref, bs_ref):
    bidx = pl.program_id(0)
    rem = id_ref[bidx] % 8
    sel = jax.lax.broadcasted_iota(jnp.int32, (8, 3), 0) == rem
    ws_ref[0] = jnp.sum(jnp.where(sel, w_ref[0], 0.0), axis=0, keepdims=True)
    bs_ref[0] = jnp.sum(jnp.where(sel, b_ref[0], 0.0), axis=0, keepdims=True)


def _make_affine(samples_per_block, n_chan):
    def _affine_kernel(ws_ref, bs_ref, img_ref, out_ref):
        bidx = pl.program_id(0)
        for s in range(samples_per_block):
            sample = bidx * samples_per_block + s
            for c in range(n_chan):
                wv = ws_ref[sample * n_chan + c]
                bv = bs_ref[sample * n_chan + c]
                out_ref[s, c] = img_ref[s, c] * wv + bv

    return _affine_kernel


def kernel(image, camindex, idindex, w, b):
    B, C, H, W = image.shape
    ws, bs = pl.pallas_call(
        _gather_kernel,
        grid_spec=pltpu.PrefetchScalarGridSpec(
            num_scalar_prefetch=2,
            grid=(B,),
            in_specs=[
                pl.BlockSpec((1, 8, 3), lambda bi, cam, idx: (cam[bi], idx[bi] // 8, 0)),
                pl.BlockSpec((1, 8, 3), lambda bi, cam, idx: (cam[bi], idx[bi] // 8, 0)),
            ],
            out_specs=[
                pl.BlockSpec((1, 1, 3), lambda bi, cam, idx: (bi, 0, 0)),
                pl.BlockSpec((1, 1, 3), lambda bi, cam, idx: (bi, 0, 0)),
            ],
        ),
        out_shape=[jax.ShapeDtypeStruct((B, 1, 3), jnp.float32)] * 2,
    )(camindex, idindex, w, b)

    sb = _SAMPLES_PER_BLOCK
    hb = H // _H_SPLIT
    ws_flat = ws.reshape(B * C)
    bs_flat = bs.reshape(B * C)
    out = pl.pallas_call(
        _make_affine(sb, C),
        grid_spec=pltpu.PrefetchScalarGridSpec(
            num_scalar_prefetch=2,
            grid=(B // sb, _H_SPLIT),
            in_specs=[
                pl.BlockSpec((sb, C, hb, W), lambda bi, hi, wsr, bsr: (bi, 0, hi, 0)),
            ],
            out_specs=pl.BlockSpec((sb, C, hb, W), lambda bi, hi, wsr, bsr: (bi, 0, hi, 0)),
        ),
        out_shape=jax.ShapeDtypeStruct(image.shape, image.dtype),
    )(ws_flat, bs_flat, image)
    return out
